# baseline (speedup 1.0000x reference)
import functools

import jax
import jax.numpy as jnp
from jax import lax
from jax.experimental import pallas as pl
from jax.experimental.pallas import tpu as pltpu
from jax.experimental.pallas import tpu_sc as plsc

B, T, D = 4096, 200, 64
NW = 32
TG = T // 8
SKEW = 65


def _body(x4_hbm, tab_hbm, pos_hbm, out_hbm, idx_v, pos_v, bufg0, bufg1,
          bufs0, bufs1, bufo0, bufo1, gsem, wsem0, wsem1):
    wid = lax.axis_index("s") * 2 + lax.axis_index("c")

    pltpu.sync_copy(x4_hbm.at[wid], idx_v)
    pltpu.sync_copy(pos_hbm, pos_v)

    iota = lax.iota(jnp.int32, 16)

    def gather(t, buf):
        return pltpu.make_async_copy(
            tab_hbm.at[idx_v.at[lax.shift_right_logical(t, 3),
                                lax.bitwise_and(t, 7)]],
            buf, gsem)

    def step(t, carry):
        p = lax.rem(t, 2)

        def run(bufg, obufg, bufs, bufo, wsem):
            gather(t, bufg).wait()

            @pl.when(t + 1 < T)
            def _():
                gather(t + 1, obufg).start()

            posr = [pos_v[t, pl.ds(16 * c, 16)] for c in range(4)]

            def tok_body(tok, c2):
                for c in range(4):
                    sl = pl.ds(16 * c, 16)
                    bufs[tok, sl] = bufg[tok, sl] + posr[c]
                return c2

            lax.fori_loop(0, 128, tok_body, 0)

            @pl.when(t >= 2)
            def _():
                for dg in range(8):
                    pltpu.make_async_copy(
                        bufo.at[pl.ds(dg * 8, 8)],
                        out_hbm.at[t - 2, dg, wid], wsem).wait()

            def tg_body(tg, c2):
                tokv = tg * 16 + iota
                for d in range(D):
                    v = plsc.load_gather(
                        bufs, [tokv, jnp.full((16,), d, dtype=jnp.int32)])
                    bufo[d, pl.ds(tg * 16, 16)] = v
                return c2

            lax.fori_loop(0, 8, tg_body, 0)

            for dg in range(8):
                pltpu.async_copy(bufo.at[pl.ds(dg * 8, 8)],
                                 out_hbm.at[t, dg, wid], wsem)

        @pl.when(p == 0)
        def _():
            run(bufg0, bufg1, bufs0, bufo0, wsem0)

        @pl.when(p == 1)
        def _():
            run(bufg1, bufg0, bufs1, bufo1, wsem1)

        return carry

    gather(0, bufg0).start()
    lax.fori_loop(0, T, step, 0)

    for t, (bufo, wsem) in ((T - 2, (bufo0, wsem0)), (T - 1, (bufo1, wsem1))):
        for dg in range(8):
            pltpu.make_async_copy(bufo.at[pl.ds(dg * 8, 8)],
                                  out_hbm.at[t, dg, wid], wsem).wait()


@jax.jit
def kernel(x, token_table, pos_table):
    x4 = x.T.reshape(TG, 8, NW, 128).transpose(2, 0, 1, 3)

    mesh = plsc.VectorSubcoreMesh(core_axis_name="c", subcore_axis_name="s")
    k = functools.partial(
        pl.kernel,
        out_type=jax.ShapeDtypeStruct((T, 8, NW, 8, 128), jnp.float32),
        mesh=mesh,
        scratch_types=[
            pltpu.VMEM((TG, 8, 128), jnp.int32),
            pltpu.VMEM((T, D), jnp.float32),
            pltpu.VMEM((128, D), jnp.float32),
            pltpu.VMEM((128, D), jnp.float32),
            pltpu.VMEM((128, SKEW), jnp.float32),
            pltpu.VMEM((128, SKEW), jnp.float32),
            pltpu.VMEM((D, 128), jnp.float32),
            pltpu.VMEM((D, 128), jnp.float32),
            pltpu.SemaphoreType.DMA,
            pltpu.SemaphoreType.DMA,
            pltpu.SemaphoreType.DMA,
        ],
        compiler_params=pltpu.CompilerParams(
            use_tc_tiling_on_sc=False, needs_layout_passes=False),
    )(_body)
    out5 = k(x4, token_table, pos_table)
    return out5.transpose(2, 4, 0, 1, 3).reshape(B, T, D)


# speedup vs baseline: 1.9589x; 1.9589x over previous
import functools

import jax
import jax.numpy as jnp
from jax import lax
from jax.experimental import pallas as pl
from jax.experimental.pallas import tpu as pltpu
from jax.experimental.pallas import tpu_sc as plsc

B, T, D = 4096, 200, 64
NW = 32
TG = T // 8
SKEW = 65


def _body(x4_hbm, tab_hbm, pos_hbm, out_hbm, idx_v, pos_v, bufg0, bufg1,
          bufs0, bufs1, bufo0, bufo1, gsem, wsem0, wsem1):
    wid = lax.axis_index("s") * 2 + lax.axis_index("c")

    pltpu.sync_copy(x4_hbm.at[wid], idx_v)
    pltpu.sync_copy(pos_hbm, pos_v)

    base65 = lax.iota(jnp.int32, 16) * SKEW

    def gather(t, buf):
        return pltpu.make_async_copy(
            tab_hbm.at[idx_v.at[lax.shift_right_logical(t, 3),
                                lax.bitwise_and(t, 7)]],
            buf, gsem)

    def step(t, carry):
        p = lax.rem(t, 2)

        def run(bufg, obufg, bufs, bufo, wsem):
            gather(t, bufg).wait()

            @pl.when(t + 1 < T)
            def _():
                gather(t + 1, obufg).start()

            posr = [pos_v[t, pl.ds(16 * c, 16)] for c in range(4)]

            @plsc.parallel_loop(0, 128, unroll=4)
            def _(tok):
                for c in range(4):
                    bufs[pl.ds(tok * SKEW + 16 * c, 16)] = (
                        bufg[tok, pl.ds(16 * c, 16)] + posr[c])

            @pl.when(t >= 2)
            def _():
                for dg in range(8):
                    pltpu.make_async_copy(
                        bufo.at[pl.ds(dg * 1024, 1024)],
                        out_hbm.at[t - 2, dg, wid], wsem).wait()

            @plsc.parallel_loop(0, 8 * D, unroll=8)
            def _(i):
                tg = lax.shift_right_logical(i, 6)
                d = lax.bitwise_and(i, 63)
                v = plsc.load_gather(bufs, [base65 + (tg * (16 * SKEW) + d)])
                bufo[pl.ds(d * 128 + tg * 16, 16)] = v

            for dg in range(8):
                pltpu.async_copy(bufo.at[pl.ds(dg * 1024, 1024)],
                                 out_hbm.at[t, dg, wid], wsem)

        @pl.when(p == 0)
        def _():
            run(bufg0, bufg1, bufs0, bufo0, wsem0)

        @pl.when(p == 1)
        def _():
            run(bufg1, bufg0, bufs1, bufo1, wsem1)

        return carry

    gather(0, bufg0).start()
    lax.fori_loop(0, T, step, 0)

    for t, (bufo, wsem) in ((T - 2, (bufo0, wsem0)), (T - 1, (bufo1, wsem1))):
        for dg in range(8):
            pltpu.make_async_copy(bufo.at[pl.ds(dg * 1024, 1024)],
                                  out_hbm.at[t, dg, wid], wsem).wait()


@jax.jit
def kernel(x, token_table, pos_table):
    x4 = x.T.reshape(TG, 8, NW, 128).transpose(2, 0, 1, 3)

    mesh = plsc.VectorSubcoreMesh(core_axis_name="c", subcore_axis_name="s")
    k = functools.partial(
        pl.kernel,
        out_type=jax.ShapeDtypeStruct((T, 8, NW, 1024), jnp.float32),
        mesh=mesh,
        scratch_types=[
            pltpu.VMEM((TG, 8, 128), jnp.int32),
            pltpu.VMEM((T, D), jnp.float32),
            pltpu.VMEM((128, D), jnp.float32),
            pltpu.VMEM((128, D), jnp.float32),
            pltpu.VMEM((128 * SKEW,), jnp.float32),
            pltpu.VMEM((128 * SKEW,), jnp.float32),
            pltpu.VMEM((D * 128,), jnp.float32),
            pltpu.VMEM((D * 128,), jnp.float32),
            pltpu.SemaphoreType.DMA,
            pltpu.SemaphoreType.DMA,
            pltpu.SemaphoreType.DMA,
        ],
        compiler_params=pltpu.CompilerParams(
            use_tc_tiling_on_sc=False, needs_layout_passes=False),
    )(_body)
    out6 = k(x4, token_table, pos_table)
    return (out6.reshape(T, 8, NW, 8, 128)
            .transpose(2, 4, 0, 1, 3).reshape(B, T, D))


# speedup vs baseline: 2.0084x; 1.0252x over previous
import functools

import jax
import jax.numpy as jnp
from jax import lax
from jax.experimental import pallas as pl
from jax.experimental.pallas import tpu as pltpu
from jax.experimental.pallas import tpu_sc as plsc

B, T, D = 4096, 200, 64
NW = 32
TG = T // 8
SKEW = 65


def _body(x4_hbm, tab_hbm, pos_hbm, out_hbm, idx_v, pos_v, bufg0, bufg1,
          bufs0, bufs1, bufo0, bufo1, gsem0, gsem1, wsem0, wsem1):
    wid = lax.axis_index("s") * 2 + lax.axis_index("c")

    pltpu.sync_copy(x4_hbm.at[wid], idx_v)
    pltpu.sync_copy(pos_hbm, pos_v)

    base65 = lax.iota(jnp.int32, 16) * SKEW

    def gather(t, buf, sem):
        return pltpu.make_async_copy(
            tab_hbm.at[idx_v.at[lax.shift_right_logical(t, 3),
                                lax.bitwise_and(t, 7)]],
            buf, sem)

    def step(t, carry):
        p = lax.rem(t, 2)

        def run(bufg, obufg, bufs, bufo, gsem, ogsem, wsem):
            @pl.when(t + 1 < T)
            def _():
                gather(t + 1, obufg, ogsem).start()

            gather(t, bufg, gsem).wait()

            posr = [pos_v[t, pl.ds(16 * c, 16)] for c in range(4)]

            @plsc.parallel_loop(0, 128, unroll=8)
            def _(tok):
                for c in range(4):
                    bufs[pl.ds(tok * SKEW + 16 * c, 16)] = (
                        bufg[tok, pl.ds(16 * c, 16)] + posr[c])

            @pl.when(t >= 2)
            def _():
                for dg in range(8):
                    pltpu.make_async_copy(
                        bufo.at[pl.ds(dg * 1024, 1024)],
                        out_hbm.at[t - 2, dg, wid], wsem).wait()

            @plsc.parallel_loop(0, 8 * D, unroll=8)
            def _(i):
                tg = lax.shift_right_logical(i, 6)
                d = lax.bitwise_and(i, 63)
                v = plsc.load_gather(bufs, [base65 + (tg * (16 * SKEW) + d)])
                bufo[pl.ds(d * 128 + tg * 16, 16)] = v

            for dg in range(8):
                pltpu.async_copy(bufo.at[pl.ds(dg * 1024, 1024)],
                                 out_hbm.at[t, dg, wid], wsem)

        @pl.when(p == 0)
        def _():
            run(bufg0, bufg1, bufs0, bufo0, gsem0, gsem1, wsem0)

        @pl.when(p == 1)
        def _():
            run(bufg1, bufg0, bufs1, bufo1, gsem1, gsem0, wsem1)

        return carry

    gather(0, bufg0, gsem0).start()
    lax.fori_loop(0, T, step, 0)

    for t, (bufo, wsem) in ((T - 2, (bufo0, wsem0)), (T - 1, (bufo1, wsem1))):
        for dg in range(8):
            pltpu.make_async_copy(bufo.at[pl.ds(dg * 1024, 1024)],
                                  out_hbm.at[t, dg, wid], wsem).wait()


@jax.jit
def kernel(x, token_table, pos_table):
    x4 = x.T.reshape(TG, 8, NW, 128).transpose(2, 0, 1, 3)

    mesh = plsc.VectorSubcoreMesh(core_axis_name="c", subcore_axis_name="s")
    k = functools.partial(
        pl.kernel,
        out_type=jax.ShapeDtypeStruct((T, 8, NW, 1024), jnp.float32),
        mesh=mesh,
        scratch_types=[
            pltpu.VMEM((TG, 8, 128), jnp.int32),
            pltpu.VMEM((T, D), jnp.float32),
            pltpu.VMEM((128, D), jnp.float32),
            pltpu.VMEM((128, D), jnp.float32),
            pltpu.VMEM((128 * SKEW,), jnp.float32),
            pltpu.VMEM((128 * SKEW,), jnp.float32),
            pltpu.VMEM((D * 128,), jnp.float32),
            pltpu.VMEM((D * 128,), jnp.float32),
            pltpu.SemaphoreType.DMA,
            pltpu.SemaphoreType.DMA,
            pltpu.SemaphoreType.DMA,
            pltpu.SemaphoreType.DMA,
        ],
        compiler_params=pltpu.CompilerParams(
            use_tc_tiling_on_sc=False, needs_layout_passes=False),
    )(_body)
    out6 = k(x4, token_table, pos_table)
    return (out6.reshape(T, 8, NW, 8, 128)
            .transpose(2, 4, 0, 1, 3).reshape(B, T, D))


# speedup vs baseline: 2.0262x; 1.0089x over previous
import functools

import jax
import jax.numpy as jnp
from jax import lax
from jax.experimental import pallas as pl
from jax.experimental.pallas import tpu as pltpu
from jax.experimental.pallas import tpu_sc as plsc

B, T, D = 4096, 200, 64
NW = 32
TG = T // 8
SKEW = 65


def _body(x4_hbm, tab_hbm, pos_hbm, out_hbm, idx_v, pos_v, bufg0, bufg1,
          bufs0, bufs1, bufo0, bufo1, gsem0, gsem1, wsem0, wsem1):
    wid = lax.axis_index("s") * 2 + lax.axis_index("c")

    pltpu.sync_copy(x4_hbm.at[wid], idx_v)
    pltpu.sync_copy(pos_hbm, pos_v)

    base65 = lax.iota(jnp.int32, 16) * SKEW

    def gather(t, buf, sem):
        return pltpu.make_async_copy(
            tab_hbm.at[idx_v.at[lax.shift_right_logical(t, 3),
                                lax.bitwise_and(t, 7)]],
            buf, sem)

    def step(t, carry):
        p = lax.rem(t, 2)

        def run(bufg, obufg, bufs, bufo, gsem, ogsem, wsem):
            @pl.when(t + 1 < T)
            def _():
                gather(t + 1, obufg, ogsem).start()

            gather(t, bufg, gsem).wait()

            posr = [pos_v[t, pl.ds(16 * c, 16)] for c in range(4)]

            @plsc.parallel_loop(0, 128, unroll=8)
            def _(tok):
                for c in range(4):
                    bufs[pl.ds(tok * SKEW + 16 * c, 16)] = (
                        bufg[tok, pl.ds(16 * c, 16)] + posr[c])

            @pl.when(t >= 2)
            def _():
                for dg in range(8):
                    pltpu.make_async_copy(
                        bufo.at[pl.ds(dg * 1024, 1024)],
                        out_hbm.at[t - 2, dg, wid], wsem).wait()

            @plsc.parallel_loop(0, 8 * D, unroll=16)
            def _(i):
                tg = lax.shift_right_logical(i, 6)
                d = lax.bitwise_and(i, 63)
                v = plsc.load_gather(bufs, [base65 + (tg * (16 * SKEW) + d)])
                bufo[pl.ds(d * 128 + tg * 16, 16)] = v

            for dg in range(8):
                pltpu.async_copy(bufo.at[pl.ds(dg * 1024, 1024)],
                                 out_hbm.at[t, dg, wid], wsem)

        @pl.when(p == 0)
        def _():
            run(bufg0, bufg1, bufs0, bufo0, gsem0, gsem1, wsem0)

        @pl.when(p == 1)
        def _():
            run(bufg1, bufg0, bufs1, bufo1, gsem1, gsem0, wsem1)

        return carry

    gather(0, bufg0, gsem0).start()
    lax.fori_loop(0, T, step, 0)

    for t, (bufo, wsem) in ((T - 2, (bufo0, wsem0)), (T - 1, (bufo1, wsem1))):
        for dg in range(8):
            pltpu.make_async_copy(bufo.at[pl.ds(dg * 1024, 1024)],
                                  out_hbm.at[t, dg, wid], wsem).wait()


@jax.jit
def kernel(x, token_table, pos_table):
    x4 = x.T.reshape(TG, 8, NW, 128).transpose(2, 0, 1, 3)

    mesh = plsc.VectorSubcoreMesh(core_axis_name="c", subcore_axis_name="s")
    k = functools.partial(
        pl.kernel,
        out_type=jax.ShapeDtypeStruct((T, 8, NW, 1024), jnp.float32),
        mesh=mesh,
        scratch_types=[
            pltpu.VMEM((TG, 8, 128), jnp.int32),
            pltpu.VMEM((T, D), jnp.float32),
            pltpu.VMEM((128, D), jnp.float32),
            pltpu.VMEM((128, D), jnp.float32),
            pltpu.VMEM((128 * SKEW,), jnp.float32),
            pltpu.VMEM((128 * SKEW,), jnp.float32),
            pltpu.VMEM((D * 128,), jnp.float32),
            pltpu.VMEM((D * 128,), jnp.float32),
            pltpu.SemaphoreType.DMA,
            pltpu.SemaphoreType.DMA,
            pltpu.SemaphoreType.DMA,
            pltpu.SemaphoreType.DMA,
        ],
        compiler_params=pltpu.CompilerParams(
            use_tc_tiling_on_sc=False, needs_layout_passes=False),
    )(_body)
    out6 = k(x4, token_table, pos_table)
    return (out6.reshape(T, 8, NW, 8, 128)
            .transpose(2, 4, 0, 1, 3).reshape(B, T, D))
